# baseline (speedup 1.0000x reference)
import functools

import jax
import jax.numpy as jnp
from jax import lax
from jax.experimental import pallas as pl
from jax.experimental.pallas import tpu as pltpu
from jax.experimental.pallas import tpu_sc as plsc

LANE = 128
NC = 2
NS = 16
NW = NC * NS


def _sc_mesh():
    return plsc.VectorSubcoreMesh(core_axis_name="c", subcore_axis_name="s")


def _make_deg_kernel(n, nch):
    npad = -(-n // (NS * LANE)) * (NS * LANE)
    cpt = npad // NS

    @functools.partial(
        pl.kernel,
        out_type=jax.ShapeDtypeStruct((NC * npad,), jnp.float32),
        mesh=_sc_mesh(),
        scratch_types=[
            pltpu.VMEM((nch, LANE), jnp.int32),
            pltpu.VMEM((nch, LANE), jnp.float32),
            pltpu.VMEM((cpt,), jnp.float32),
            pltpu.VMEM_SHARED((npad,), jnp.float32),
        ],
        compiler_params=pltpu.CompilerParams(needs_layout_passes=False),
    )
    def deg_kernel(colb, wb, degp, col_v, w_v, zbuf, deg_s):
        cid = lax.axis_index("c")
        sid = lax.axis_index("s")
        wid = sid * NC + cid
        zeros = jnp.zeros((16,), jnp.float32)

        def zbody(i, carry):
            zbuf[pl.ds(i * 16, 16)] = zeros
            return carry

        lax.fori_loop(0, cpt // 16, zbody, 0)
        pltpu.sync_copy(zbuf, deg_s.at[pl.ds(sid * cpt, cpt)])
        plsc.subcore_barrier()

        pltpu.sync_copy(colb.at[wid], col_v)
        pltpu.sync_copy(wb.at[wid], w_v)

        def body(j, carry):
            pltpu.sync_copy(w_v.at[j], deg_s.at[col_v.at[j]], add=True)
            return carry

        lax.fori_loop(0, nch, body, 0)
        plsc.subcore_barrier()
        pltpu.sync_copy(
            deg_s.at[pl.ds(sid * cpt, cpt)],
            degp.at[pl.ds(cid * npad + sid * cpt, cpt)],
        )

    return deg_kernel


def _splat(vec, lane):
    idx = jnp.full((16, 1), lane, jnp.int32)
    return lax.gather(
        vec, idx,
        lax.GatherDimensionNumbers(
            offset_dims=(), collapsed_slice_dims=(0,), start_index_map=(0,)),
        (1,), mode=lax.GatherScatterMode.PROMISE_IN_BOUNDS)


def _make_mp_kernel(n, d, nch):
    nr = -(-n // (NS * 8)) * (NS * 8)
    rpt = nr // NS
    RB = 4
    assert nch % RB == 0

    @functools.partial(
        pl.kernel,
        out_type=jax.ShapeDtypeStruct((NC, nr, d), jnp.float32),
        mesh=_sc_mesh(),
        scratch_types=[
            [pltpu.VMEM((3, LANE), jnp.int32) for _ in range(RB)],
            [pltpu.SemaphoreType.DMA for _ in range(RB)],
            pltpu.VMEM((LANE, 128), jnp.float32),
            pltpu.VMEM((LANE, 128), jnp.float32),
            pltpu.VMEM_SHARED((nr, 128), jnp.float32),
            pltpu.SemaphoreType.DMA,
            pltpu.SemaphoreType.DMA,
            pltpu.SemaphoreType.DMA,
            pltpu.SemaphoreType.DMA,
        ],
        compiler_params=pltpu.CompilerParams(needs_layout_passes=False),
    )
    def mp_kernel(hp, eb, agg, ring, esems, gb0, gb1, acc, sem0, sem1,
                  ssem0, ssem1):
        cid = lax.axis_index("c")
        sid = lax.axis_index("s")
        wid = sid * NC + cid
        zeros = jnp.zeros((16,), jnp.float32)
        bufs = (gb0, gb1)
        sems = (sem0, sem1)
        ssems = (ssem0, ssem1)

        def edge_start(j, s):
            pltpu.async_copy(eb.at[wid, j], ring[s], esems[s])

        def edge_wait(j, s):
            pltpu.make_async_copy(eb.at[wid, j], ring[s], esems[s]).wait()

        def gather_start(j, s, b):
            pltpu.async_copy(hp.at[ring[s].at[0]], bufs[b], sems[b])

        def gather_wait(j, s, b):
            pltpu.make_async_copy(hp.at[ring[s].at[0]], bufs[b],
                                  sems[b]).wait()

        def zb(i, carry):
            for r in range(d // 16):
                gb0[i, pl.ds(r * 16, 16)] = zeros
            return carry

        lax.fori_loop(0, LANE, zb, 0)
        nz = rpt // LANE + (1 if rpt % LANE else 0)
        off = 0
        for k in range(nz):
            step = min(LANE, rpt - k * LANE)
            pltpu.sync_copy(
                gb0.at[pl.ds(0, step)],
                acc.at[pl.ds(sid * rpt + off, step)],
            )
            off += step
        plsc.subcore_barrier()

        for s in range(RB - 1):
            edge_start(s, s)
        edge_wait(0, 0)
        gather_start(0, 0, 0)

        def scatter_start(s, b):
            pltpu.async_copy(bufs[b], acc.at[ring[s].at[1]], ssems[b],
                             add=True)

        def scatter_wait(s, b):
            pltpu.make_async_copy(bufs[b], acc.at[ring[s].at[1]],
                                  ssems[b]).wait()

        def outer(jj, carry):
            for t in range(RB):
                j = jj * RB + t
                s = t
                b = t % 2
                buf = bufs[b]
                gather_wait(j, s, b)

                @pl.when(j >= 1)
                def _():
                    scatter_wait((s + RB - 1) % RB, 1 - b)

                @pl.when(j + 1 < nch)
                def _():
                    edge_wait(j + 1, (s + 1) % RB)
                    gather_start(j + 1, (s + 1) % RB, 1 - b)

                @pl.when(j + RB - 1 < nch)
                def _():
                    edge_start(j + RB - 1, (s + RB - 1) % RB)

                def grp(g, carry2):
                    base = g * 16
                    wv = plsc.bitcast(ring[s][2, pl.ds(base, 16)],
                                      jnp.float32)
                    for i in range(16):
                        ws = _splat(wv, i)
                        e = base + i
                        for r in range(d // 16):
                            buf[e, pl.ds(r * 16, 16)] = (
                                buf[e, pl.ds(r * 16, 16)] * ws
                            )
                    return carry2

                lax.fori_loop(0, LANE // 16, grp, 0)

                scatter_start(s, b)
            return carry

        lax.fori_loop(0, nch // RB, outer, 0)
        scatter_wait(RB - 1, (nch - 1) % 2)
        plsc.subcore_barrier()
        pltpu.sync_copy(
            acc.at[pl.ds(sid * rpt, rpt)],
            agg.at[cid, pl.ds(sid * rpt, rpt)],
        )

    return mp_kernel


def _prep_body(degt_ref, x_ref, w1_ref, dinv_ref, hp_ref):
    deg = jnp.sum(degt_ref[...], axis=1, keepdims=True) + 1.0
    dinv = lax.rsqrt(deg)
    dinv_ref[...] = dinv
    hp_ref[...] = (
        jnp.dot(x_ref[...], w1_ref[...], preferred_element_type=jnp.float32)
        * dinv
    )


def _mid_body(agg_ref, hp_ref, dinv_ref, g_ref, be_ref, a_ref, wn_ref, out_ref):
    nn = hp_ref.shape[0]
    dinv = dinv_ref[...]
    pre = (agg_ref[0, :nn] + agg_ref[1, :nn] + hp_ref[...]) * dinv
    m = jnp.mean(pre, axis=0, keepdims=True)
    c = pre - m
    v = jnp.mean(c * c, axis=0, keepdims=True)
    y = g_ref[...] * (c * lax.rsqrt(v + 1e-5)) + be_ref[...]
    y = jnp.where(y >= 0, y, a_ref[...] * y)
    out_ref[...] = (
        jnp.dot(y, wn_ref[...], preferred_element_type=jnp.float32) * dinv
    )


def _final_body(agg_ref, hp_ref, dinv_ref, g_ref, be_ref, a_ref, out_ref):
    nn = hp_ref.shape[0]
    pre = (agg_ref[0, :nn] + agg_ref[1, :nn] + hp_ref[...]) * dinv_ref[...]
    m = jnp.mean(pre, axis=0, keepdims=True)
    c = pre - m
    v = jnp.mean(c * c, axis=0, keepdims=True)
    y = g_ref[...] * (c * lax.rsqrt(v + 1e-5)) + be_ref[...]
    out_ref[...] = jnp.where(y >= 0, y, a_ref[...] * y)


def kernel(x, edge_index, edge_weight, W1, b1, g1, be1, a1,
           W2, b2, g2, be2, a2, W3, b3, g3, be3, a3):
    n, din = x.shape
    dh = W1.shape[1]
    dout = W3.shape[1]
    e = edge_weight.shape[0]
    assert din == 128 and dh == 128 and dout == 128 and n % NS == 0

    row = edge_index[0]
    col = edge_index[1]

    ept = -(-e // NW)
    ept = -(-ept // (4 * LANE)) * (4 * LANE)
    nch = ept // LANE
    ep = NW * ept
    pad = ep - e
    if pad:
        fill = (jnp.arange(pad, dtype=jnp.int32) * 7) % n
        row = jnp.concatenate([row, fill])
        col = jnp.concatenate([col, fill])
        ew = jnp.concatenate([edge_weight, jnp.zeros((pad,), jnp.float32)])
    else:
        ew = edge_weight
    rowb = row.reshape(NW, nch, LANE)
    colb = col.reshape(NW, nch, LANE)
    wb = ew.reshape(NW, nch, LANE)
    eb = jnp.stack(
        [rowb, colb, jax.lax.bitcast_convert_type(wb, jnp.int32)], axis=2)

    degf = _make_deg_kernel(n, nch)(colb, wb)
    degp = degf.reshape(NC, -1)[:, :n]

    prep = pl.pallas_call(
        _prep_body,
        out_shape=[
            jax.ShapeDtypeStruct((n, 1), jnp.float32),
            jax.ShapeDtypeStruct((n, dh), jnp.float32),
        ],
    )
    dinv, hp = prep(degp.T, x, W1)

    mp = _make_mp_kernel(n, dh, nch)
    mid = pl.pallas_call(
        _mid_body,
        out_shape=jax.ShapeDtypeStruct((n, dh), jnp.float32),
    )
    fin = pl.pallas_call(
        _final_body,
        out_shape=jax.ShapeDtypeStruct((n, dout), jnp.float32),
    )

    g1r, be1r, a1r = g1.reshape(1, dh), be1.reshape(1, dh), a1.reshape(1, dh)
    g2r, be2r, a2r = g2.reshape(1, dh), be2.reshape(1, dh), a2.reshape(1, dh)
    g3r, be3r, a3r = g3.reshape(1, dout), be3.reshape(1, dout), a3.reshape(1, dout)

    agg = mp(hp, eb)
    hp = mid(agg, hp, dinv, g1r, be1r, a1r, W2)
    agg = mp(hp, eb)
    hp = mid(agg, hp, dinv, g2r, be2r, a2r, W3)
    agg = mp(hp, eb)
    return fin(agg, hp, dinv, g3r, be3r, a3r)


# speedup vs baseline: 1.1149x; 1.1149x over previous
import functools

import jax
import jax.numpy as jnp
from jax import lax
from jax.experimental import pallas as pl
from jax.experimental.pallas import tpu as pltpu
from jax.experimental.pallas import tpu_sc as plsc

LANE = 128
CH = 64
NB = 4
NE = 8
NC = 2
NS = 16
NW = NC * NS


def _sc_mesh():
    return plsc.VectorSubcoreMesh(core_axis_name="c", subcore_axis_name="s")


def _make_deg_kernel(n, nch):
    npad = -(-n // (NS * LANE)) * (NS * LANE)
    cpt = npad // NS

    @functools.partial(
        pl.kernel,
        out_type=jax.ShapeDtypeStruct((NC * npad,), jnp.float32),
        mesh=_sc_mesh(),
        scratch_types=[
            pltpu.VMEM((nch, LANE), jnp.int32),
            pltpu.VMEM((nch, LANE), jnp.float32),
            pltpu.VMEM((cpt,), jnp.float32),
            pltpu.VMEM_SHARED((npad,), jnp.float32),
        ],
        compiler_params=pltpu.CompilerParams(needs_layout_passes=False),
    )
    def deg_kernel(colb, wb, degp, col_v, w_v, zbuf, deg_s):
        cid = lax.axis_index("c")
        sid = lax.axis_index("s")
        wid = sid * NC + cid
        zeros = jnp.zeros((16,), jnp.float32)

        def zbody(i, carry):
            zbuf[pl.ds(i * 16, 16)] = zeros
            return carry

        lax.fori_loop(0, cpt // 16, zbody, 0)
        pltpu.sync_copy(zbuf, deg_s.at[pl.ds(sid * cpt, cpt)])
        plsc.subcore_barrier()

        pltpu.sync_copy(colb.at[wid], col_v)
        pltpu.sync_copy(wb.at[wid], w_v)

        def body(j, carry):
            pltpu.sync_copy(w_v.at[j], deg_s.at[col_v.at[j]], add=True)
            return carry

        lax.fori_loop(0, nch, body, 0)
        plsc.subcore_barrier()
        pltpu.sync_copy(
            deg_s.at[pl.ds(sid * cpt, cpt)],
            degp.at[pl.ds(cid * npad + sid * cpt, cpt)],
        )

    return deg_kernel


def _splat(vec, lane):
    idx = jnp.full((16, 1), lane, jnp.int32)
    return lax.gather(
        vec, idx,
        lax.GatherDimensionNumbers(
            offset_dims=(), collapsed_slice_dims=(0,), start_index_map=(0,)),
        (1,), mode=lax.GatherScatterMode.PROMISE_IN_BOUNDS)


def _make_mp_kernel(n, d, nch):
    nr = -(-n // (NS * 8)) * (NS * 8)
    rpt = nr // NS
    assert nch % NE == 0

    @functools.partial(
        pl.kernel,
        out_type=jax.ShapeDtypeStruct((NC, nr, d), jnp.float32),
        mesh=_sc_mesh(),
        scratch_types=[
            [pltpu.VMEM((3, CH), jnp.int32) for _ in range(NE)],
            [pltpu.SemaphoreType.DMA for _ in range(NE)],
            [pltpu.VMEM((CH, 128), jnp.float32) for _ in range(NB)],
            [pltpu.SemaphoreType.DMA for _ in range(NB)],
            [pltpu.SemaphoreType.DMA for _ in range(NB)],
            pltpu.VMEM_SHARED((nr, 128), jnp.float32),
        ],
        compiler_params=pltpu.CompilerParams(needs_layout_passes=False),
    )
    def mp_kernel(hp, eb, agg, ring, esems, bufs, gsems, ssems, acc):
        cid = lax.axis_index("c")
        sid = lax.axis_index("s")
        wid = sid * NC + cid
        zeros = jnp.zeros((16,), jnp.float32)

        def edge_start(j, u):
            pltpu.async_copy(eb.at[wid, j], ring[u], esems[u])

        def edge_wait(j, u):
            pltpu.make_async_copy(eb.at[wid, j], ring[u], esems[u]).wait()

        def gather_start(j, u, b):
            pltpu.async_copy(hp.at[ring[u].at[0]], bufs[b], gsems[b])

        def gather_wait(j, u, b):
            pltpu.make_async_copy(hp.at[ring[u].at[0]], bufs[b],
                                  gsems[b]).wait()

        def scatter_start(u, b):
            pltpu.async_copy(bufs[b], acc.at[ring[u].at[1]], ssems[b],
                             add=True)

        def scatter_wait(u, b):
            pltpu.make_async_copy(bufs[b], acc.at[ring[u].at[1]],
                                  ssems[b]).wait()

        def zb(i, carry):
            for r in range(d // 16):
                bufs[0][i, pl.ds(r * 16, 16)] = zeros
            return carry

        lax.fori_loop(0, CH, zb, 0)
        off = 0
        while off < rpt:
            step = min(CH, rpt - off)
            pltpu.sync_copy(
                bufs[0].at[pl.ds(0, step)],
                acc.at[pl.ds(sid * rpt + off, step)],
            )
            off += step
        plsc.subcore_barrier()

        for p in range(NE - 3):
            edge_start(p, p)
        edge_wait(0, 0)
        gather_start(0, 0, 0)
        edge_wait(1, 1)
        gather_start(1, 1, 1)

        def outer(q, carry):
            for t in range(NE):
                j = q * NE + t
                u = t
                b = t % NB

                @pl.when(j >= 2)
                def _():
                    scatter_wait((u + NE - 2) % NE, (b + 2) % NB)

                @pl.when(j + NE - 3 < nch)
                def _():
                    edge_start(j + NE - 3, (u + NE - 3) % NE)

                @pl.when(j + 2 < nch)
                def _():
                    edge_wait(j + 2, (u + 2) % NE)
                    gather_start(j + 2, (u + 2) % NE, (b + 2) % NB)

                gather_wait(j, u, b)
                buf = bufs[b]

                def grp(g, carry2):
                    base = g * 16
                    wv = plsc.bitcast(ring[u][2, pl.ds(base, 16)],
                                      jnp.float32)
                    for i in range(16):
                        ws = _splat(wv, i)
                        e = base + i
                        for r in range(d // 16):
                            buf[e, pl.ds(r * 16, 16)] = (
                                buf[e, pl.ds(r * 16, 16)] * ws
                            )
                    return carry2

                lax.fori_loop(0, CH // 16, grp, 0)

                scatter_start(u, b)
            return carry

        lax.fori_loop(0, nch // NE, outer, 0)
        scatter_wait(NE - 2, (nch - 2) % NB)
        scatter_wait(NE - 1, (nch - 1) % NB)
        plsc.subcore_barrier()
        pltpu.sync_copy(
            acc.at[pl.ds(sid * rpt, rpt)],
            agg.at[cid, pl.ds(sid * rpt, rpt)],
        )

    return mp_kernel


def _prep_body(degt_ref, x_ref, w1_ref, dinv_ref, hp_ref):
    deg = jnp.sum(degt_ref[...], axis=1, keepdims=True) + 1.0
    dinv = lax.rsqrt(deg)
    dinv_ref[...] = dinv
    hp_ref[...] = (
        jnp.dot(x_ref[...], w1_ref[...], preferred_element_type=jnp.float32)
        * dinv
    )


def _mid_body(agg_ref, hp_ref, dinv_ref, g_ref, be_ref, a_ref, wn_ref,
              out_ref):
    nn = hp_ref.shape[0]
    dinv = dinv_ref[...]
    pre = (agg_ref[0, :nn] + agg_ref[1, :nn] + hp_ref[...]) * dinv
    m = jnp.mean(pre, axis=0, keepdims=True)
    c = pre - m
    v = jnp.mean(c * c, axis=0, keepdims=True)
    y = g_ref[...] * (c * lax.rsqrt(v + 1e-5)) + be_ref[...]
    y = jnp.where(y >= 0, y, a_ref[...] * y)
    out_ref[...] = (
        jnp.dot(y, wn_ref[...], preferred_element_type=jnp.float32) * dinv
    )


def _final_body(agg_ref, hp_ref, dinv_ref, g_ref, be_ref, a_ref, out_ref):
    nn = hp_ref.shape[0]
    pre = (agg_ref[0, :nn] + agg_ref[1, :nn] + hp_ref[...]) * dinv_ref[...]
    m = jnp.mean(pre, axis=0, keepdims=True)
    c = pre - m
    v = jnp.mean(c * c, axis=0, keepdims=True)
    y = g_ref[...] * (c * lax.rsqrt(v + 1e-5)) + be_ref[...]
    out_ref[...] = jnp.where(y >= 0, y, a_ref[...] * y)


def kernel(x, edge_index, edge_weight, W1, b1, g1, be1, a1,
           W2, b2, g2, be2, a2, W3, b3, g3, be3, a3):
    n, din = x.shape
    dh = W1.shape[1]
    dout = W3.shape[1]
    e = edge_weight.shape[0]
    assert din == 128 and dh == 128 and dout == 128 and n % NS == 0

    row = edge_index[0]
    col = edge_index[1]

    ept = -(-e // NW)
    ept = -(-ept // (NE * CH)) * (NE * CH)
    nch = ept // CH
    nchd = ept // LANE
    pad = NW * ept - e
    fill = (jnp.arange(pad, dtype=jnp.int32) * 7) % n
    row = jnp.concatenate([row, fill])
    col = jnp.concatenate([col, fill])
    ew = jnp.concatenate([edge_weight, jnp.zeros((pad,), jnp.float32)])
    cold = col.reshape(NW, nchd, LANE)
    wd = ew.reshape(NW, nchd, LANE)
    eb = jnp.stack(
        [row.reshape(NW, nch, CH), col.reshape(NW, nch, CH),
         jax.lax.bitcast_convert_type(ew, jnp.int32).reshape(NW, nch, CH)],
        axis=2)

    nr = -(-n // (NS * 8)) * (NS * 8)

    degf = _make_deg_kernel(n, nchd)(cold, wd)
    degp = degf.reshape(NC, -1)[:, :n]

    prep = pl.pallas_call(
        _prep_body,
        out_shape=[
            jax.ShapeDtypeStruct((n, 1), jnp.float32),
            jax.ShapeDtypeStruct((n, dh), jnp.float32),
        ],
    )
    dinv, hp = prep(degp.T, x, W1)

    mp = _make_mp_kernel(n, dh, nch)
    mid = pl.pallas_call(
        _mid_body,
        out_shape=jax.ShapeDtypeStruct((n, dh), jnp.float32),
    )
    fin = pl.pallas_call(
        _final_body,
        out_shape=jax.ShapeDtypeStruct((n, dout), jnp.float32),
    )

    g1r, be1r, a1r = g1.reshape(1, dh), be1.reshape(1, dh), a1.reshape(1, dh)
    g2r, be2r, a2r = g2.reshape(1, dh), be2.reshape(1, dh), a2.reshape(1, dh)
    g3r, be3r, a3r = g3.reshape(1, dout), be3.reshape(1, dout), a3.reshape(1, dout)

    agg = mp(hp, eb)
    hp = mid(agg, hp, dinv, g1r, be1r, a1r, W2)
    agg = mp(hp, eb)
    hp = mid(agg, hp, dinv, g2r, be2r, a2r, W3)
    agg = mp(hp, eb)
    return fin(agg, hp, dinv, g3r, be3r, a3r)
